# Initial kernel scaffold; baseline (speedup 1.0000x reference)
import functools

import jax
import jax.numpy as jnp
from jax import lax
from jax.experimental import pallas as pl
from jax.experimental.pallas import tpu as pltpu
from jax.experimental.pallas import tpu_sc as plsc

D = 128
LANES = 16
NC = 2
NS = 16
NW = NC * NS
CHUNK = 128
D_SUB = D // LANES


def _sc_spmm(cols, rows, vals, embeds, n_chunks):
    n = embeds.shape[0]
    rows_per_tile = n // NS

    mesh = plsc.VectorSubcoreMesh(core_axis_name="c", subcore_axis_name="s")

    @functools.partial(
        pl.kernel,
        mesh=mesh,
        out_type=jax.ShapeDtypeStruct((NC, n, D), jnp.float32),
        scratch_types=[
            pltpu.VMEM((n_chunks, CHUNK), jnp.int32),
            pltpu.VMEM((n_chunks, CHUNK), jnp.int32),
            pltpu.VMEM((n_chunks, CHUNK), jnp.float32),
            pltpu.VMEM((CHUNK, D), jnp.float32),
            pltpu.VMEM_SHARED((n, D), jnp.float32),
            pltpu.SemaphoreType.DMA,
        ],
    )
    def k(cols_hbm, rows_hbm, vals_hbm, embeds_hbm, out_hbm,
          cols_v, rows_v, vals_v, gbuf, accum, sem):
        c = lax.axis_index("c")
        s = lax.axis_index("s")
        wid = c * NS + s

        def zero_row(i, carry):
            for d in range(D_SUB):
                gbuf[i, pl.ds(d * LANES, LANES)] = jnp.zeros(
                    (LANES,), jnp.float32)
            return carry
        lax.fori_loop(0, CHUNK, zero_row, 0)

        r0 = s * rows_per_tile
        full, rem = divmod(rows_per_tile, CHUNK)
        for b in range(full):
            pltpu.sync_copy(gbuf, accum.at[pl.ds(r0 + b * CHUNK, CHUNK)])
        if rem:
            pltpu.sync_copy(gbuf.at[pl.ds(0, rem)],
                            accum.at[pl.ds(r0 + full * CHUNK, rem)])
        plsc.subcore_barrier()

        pltpu.sync_copy(cols_hbm.at[wid], cols_v)
        pltpu.sync_copy(rows_hbm.at[wid], rows_v)
        pltpu.sync_copy(vals_hbm.at[wid], vals_v)

        def chunk_body(t, carry):
            pltpu.async_copy(embeds_hbm.at[cols_v.at[t]], gbuf, sem).wait()

            def scale_edge(e, inner):
                v = jnp.full((LANES,), vals_v[t, e], dtype=jnp.float32)
                for d in range(D_SUB):
                    sl = pl.ds(d * LANES, LANES)
                    gbuf[e, sl] = gbuf[e, sl] * v
                return inner
            lax.fori_loop(0, CHUNK, scale_edge, 0)

            pltpu.sync_copy(gbuf, accum.at[rows_v.at[t]], add=True)
            return carry
        lax.fori_loop(0, n_chunks, chunk_body, 0)

        plsc.subcore_barrier()
        pltpu.sync_copy(accum.at[pl.ds(r0, rows_per_tile)],
                        out_hbm.at[c, pl.ds(r0, rows_per_tile)])

    return k(cols, rows, vals, embeds)


def _combine_body(p_ref, o_ref):
    o_ref[...] = p_ref[0] + p_ref[1]


def _combine(partials):
    n, d = partials.shape[1], partials.shape[2]
    blk = 1000
    return pl.pallas_call(
        _combine_body,
        grid=(n // blk,),
        in_specs=[pl.BlockSpec((NC, blk, d), lambda i: (0, i, 0))],
        out_specs=pl.BlockSpec((blk, d), lambda i: (i, 0)),
        out_shape=jax.ShapeDtypeStruct((n, d), jnp.float32),
    )(partials)


@jax.jit
def kernel(edge_index, values, embeds):
    n = embeds.shape[0]
    e = values.shape[0]
    rows = edge_index[0].astype(jnp.int32)
    cols = edge_index[1].astype(jnp.int32)
    vals = values.astype(jnp.float32)

    per_tile = NW * CHUNK
    n_chunks = -(-e // per_tile)
    e_pad = n_chunks * per_tile
    pad = e_pad - e
    if pad:
        pad_idx = (jnp.arange(pad, dtype=jnp.int32) * 17) % n
        rows = jnp.concatenate([rows, pad_idx])
        cols = jnp.concatenate([cols, pad_idx])
        vals = jnp.concatenate([vals, jnp.zeros((pad,), jnp.float32)])

    rows = rows.reshape(NW, n_chunks, CHUNK)
    cols = cols.reshape(NW, n_chunks, CHUNK)
    vals = vals.reshape(NW, n_chunks, CHUNK)

    partials = _sc_spmm(cols, rows, vals, embeds, n_chunks)
    return _combine(partials)


# speedup vs baseline: 7.6279x; 7.6279x over previous
import functools

import jax
import jax.numpy as jnp
from jax import lax
from jax.experimental import pallas as pl
from jax.experimental.pallas import tpu as pltpu
from jax.experimental.pallas import tpu_sc as plsc

D = 128
LANES = 16
NC = 2
NS = 16
NW = NC * NS
CHUNK = 128
D_SUB = D // LANES


def _sc_spmm(cols, rows, vals, embeds, n_chunks):
    n_real = embeds.shape[0]
    rows_per_tile = -(-n_real // (NS * 8)) * 8
    n = rows_per_tile * NS

    mesh = plsc.VectorSubcoreMesh(core_axis_name="c", subcore_axis_name="s")

    @functools.partial(
        pl.kernel,
        mesh=mesh,
        out_type=jax.ShapeDtypeStruct((NC, n, D), jnp.float32),
        scratch_types=[
            pltpu.VMEM((n_chunks, CHUNK), jnp.int32),
            pltpu.VMEM((n_chunks, CHUNK), jnp.int32),
            pltpu.VMEM((n_chunks, CHUNK), jnp.float32),
            pltpu.VMEM((CHUNK, D), jnp.float32),
            pltpu.VMEM_SHARED((n, D), jnp.float32),
            pltpu.SemaphoreType.DMA,
        ],
    )
    def k(cols_hbm, rows_hbm, vals_hbm, embeds_hbm, out_hbm,
          cols_v, rows_v, vals_v, gbuf, accum, sem):
        c = lax.axis_index("c")
        s = lax.axis_index("s")
        wid = c * NS + s

        def zero_row(i, carry):
            for d in range(D_SUB):
                gbuf[i, pl.ds(d * LANES, LANES)] = jnp.zeros(
                    (LANES,), jnp.float32)
            return carry
        lax.fori_loop(0, CHUNK, zero_row, 0)

        r0 = s * rows_per_tile
        full, rem = divmod(rows_per_tile, CHUNK)
        for b in range(full):
            pltpu.sync_copy(gbuf, accum.at[pl.ds(r0 + b * CHUNK, CHUNK)])
        if rem:
            pltpu.sync_copy(gbuf.at[pl.ds(0, rem)],
                            accum.at[pl.ds(r0 + full * CHUNK, rem)])
        plsc.subcore_barrier()

        pltpu.sync_copy(cols_hbm.at[wid], cols_v)
        pltpu.sync_copy(rows_hbm.at[wid], rows_v)
        pltpu.sync_copy(vals_hbm.at[wid], vals_v)

        def chunk_body(t, carry):
            pltpu.async_copy(embeds_hbm.at[cols_v.at[t]], gbuf, sem).wait()

            def scale_group(g, inner):
                base = g * LANES
                v16 = vals_v[t, pl.ds(base, LANES)]
                for l in range(LANES):
                    vb = jnp.full((LANES,), v16[l], dtype=jnp.float32)
                    e = base + l
                    for d in range(D_SUB):
                        sl = pl.ds(d * LANES, LANES)
                        gbuf[e, sl] = gbuf[e, sl] * vb
                return inner
            lax.fori_loop(0, CHUNK // LANES, scale_group, 0)

            pltpu.sync_copy(gbuf, accum.at[rows_v.at[t]], add=True)
            return carry
        lax.fori_loop(0, n_chunks, chunk_body, 0)

        plsc.subcore_barrier()
        pltpu.sync_copy(accum.at[pl.ds(r0, rows_per_tile)],
                        out_hbm.at[c, pl.ds(r0, rows_per_tile)])

    return k(cols, rows, vals, embeds)


def _combine_body(p_ref, o_ref):
    o_ref[...] = p_ref[0] + p_ref[1]


def _combine(partials, n):
    d = partials.shape[2]
    blk = 1000
    return pl.pallas_call(
        _combine_body,
        grid=(n // blk,),
        in_specs=[pl.BlockSpec((NC, blk, d), lambda i: (0, i, 0))],
        out_specs=pl.BlockSpec((blk, d), lambda i: (i, 0)),
        out_shape=jax.ShapeDtypeStruct((n, d), jnp.float32),
    )(partials)


@jax.jit
def kernel(edge_index, values, embeds):
    n = embeds.shape[0]
    e = values.shape[0]
    rows = edge_index[0].astype(jnp.int32)
    cols = edge_index[1].astype(jnp.int32)
    vals = values.astype(jnp.float32)

    per_tile = NW * CHUNK
    n_chunks = -(-e // per_tile)
    e_pad = n_chunks * per_tile
    pad = e_pad - e
    if pad:
        pad_idx = (jnp.arange(pad, dtype=jnp.int32) * 17) % n
        rows = jnp.concatenate([rows, pad_idx])
        cols = jnp.concatenate([cols, pad_idx])
        vals = jnp.concatenate([vals, jnp.zeros((pad,), jnp.float32)])

    rows = rows.reshape(NW, n_chunks, CHUNK)
    cols = cols.reshape(NW, n_chunks, CHUNK)
    vals = vals.reshape(NW, n_chunks, CHUNK)

    partials = _sc_spmm(cols, rows, vals, embeds, n_chunks)
    return _combine(partials, n)
